# baseline (speedup 1.0000x reference)
import math

import jax
import jax.numpy as jnp
from jax import lax
from jax.experimental import pallas as pl
from jax.experimental.pallas import tpu as pltpu
from jax.experimental.pallas import tpu_sc as plsc

EMB = 64
SCALE = math.sqrt(EMB)

NC = 2
NS = 16
NW = NC * NS

BW = 128
TB = 512


def _transpose_body(tab_t_hbm, out_hbm, sb0, sb1, tb_v, tail_s, tail_t, sem0, sem1):
    vocab = tab_t_hbm.shape[1]
    n_full = vocab // TB
    per_w = (n_full + NW - 1) // NW
    wid = lax.axis_index("s") * NC + lax.axis_index("c")

    lane64 = lax.shift_left(lax.iota(jnp.int32, 16), 6)

    def v0_of(i):
        m = jnp.minimum(wid + NW * i, n_full - 1)
        return pl.multiple_of(m * TB, TB)

    def fire(i, sb, sem):
        pltpu.async_copy(tab_t_hbm.at[:, pl.ds(v0_of(i), TB)], sb, sem)

    def drain(i, sb, sem):
        pltpu.make_async_copy(
            tab_t_hbm.at[:, pl.ds(v0_of(i), TB)], sb, sem
        ).wait()

    def transpose_block(i, sb):
        def e_body(e, _):
            for g in range(TB // 16):
                x = sb[e, pl.ds(16 * g, 16)] * SCALE
                plsc.store_scatter(tb_v, [lane64 + (1024 * g + e)], x)
            return _

        lax.fori_loop(0, EMB, e_body, None)
        pltpu.sync_copy(tb_v, out_hbm.at[pl.ds(v0_of(i) * EMB, TB * EMB)])

    fire(0, sb0, sem0)

    def pair(p, _):
        i0 = 2 * p
        fire(i0 + 1, sb1, sem1)
        drain(i0, sb0, sem0)
        transpose_block(i0, sb0)
        nxt = jnp.where(i0 + 2 < per_w, i0 + 2, 0)
        fire(nxt, sb0, sem0)
        drain(i0 + 1, sb1, sem1)
        transpose_block(i0 + 1, sb1)
        return _

    lax.fori_loop(0, per_w // 2, pair, None)
    drain(0, sb0, sem0)

    v0t = n_full * TB
    pltpu.sync_copy(tab_t_hbm.at[:, pl.ds(v0t, EMB)], tail_s)

    def te_body(e, _):
        for g in range(EMB // 16):
            x = tail_s[e, pl.ds(16 * g, 16)] * SCALE
            plsc.store_scatter(tail_t, [lane64 + (1024 * g + e)], x)
        return _

    lax.fori_loop(0, EMB, te_body, None)
    pltpu.sync_copy(tail_t, out_hbm.at[pl.ds(v0t * EMB, EMB * EMB)])


def _transpose_scale(table_t):
    emb, vocab = table_t.shape
    mesh = plsc.VectorSubcoreMesh(core_axis_name="c", subcore_axis_name="s")
    return pl.kernel(
        _transpose_body,
        out_type=jax.ShapeDtypeStruct((vocab * emb,), jnp.float32),
        mesh=mesh,
        compiler_params=pltpu.CompilerParams(needs_layout_passes=False),
        scratch_types=[
            pltpu.VMEM((EMB, TB), jnp.float32),
            pltpu.VMEM((EMB, TB), jnp.float32),
            pltpu.VMEM((TB * EMB,), jnp.float32),
            pltpu.VMEM((EMB, EMB), jnp.float32),
            pltpu.VMEM((EMB * EMB,), jnp.float32),
            pltpu.SemaphoreType.DMA,
            pltpu.SemaphoreType.DMA,
        ],
    )(table_t)


def _gather_body(tokens_t_hbm, table_hbm, out_hbm, idx_all, rb0, rb1, tr, gs0, gs1):
    seq = tokens_t_hbm.shape[0]
    wid = lax.axis_index("s") * NC + lax.axis_index("c")

    pltpu.sync_copy(tokens_t_hbm.at[:, pl.ds(wid * BW, BW)], idx_all)

    lane = lax.iota(jnp.int32, 16)
    rvecs = [2 * j + lax.shift_right_logical(lane, 3) for j in range(EMB // 16)]
    cvec = lax.shift_left(jnp.bitwise_and(lane, 7), 7)

    def fire(t, rb, sem):
        pltpu.async_copy(table_hbm.at[idx_all.at[t]], rb, sem)

    def drain(t, rb, sem):
        pltpu.make_async_copy(table_hbm.at[idx_all.at[t]], rb, sem).wait()

    def process(t, rb):
        def row_body(bm, _):
            for j in range(EMB // 16):
                x = rb[bm, pl.ds(16 * j, 16)]
                plsc.store_scatter(tr, [rvecs[j], cvec + bm], x)
            return _

        lax.fori_loop(0, BW, row_body, None, unroll=2)
        for i in range(8):
            pltpu.sync_copy(tr.at[i], out_hbm.at[t, i, wid])

    fire(0, rb0, gs0)

    def pair(p, _):
        t0 = 2 * p
        fire(t0 + 1, rb1, gs1)
        drain(t0, rb0, gs0)
        process(t0, rb0)
        nxt = jnp.where(t0 + 2 < seq, t0 + 2, 0)
        fire(nxt, rb0, gs0)
        drain(t0 + 1, rb1, gs1)
        process(t0 + 1, rb1)
        return _

    lax.fori_loop(0, seq // 2, pair, None)
    drain(0, rb0, gs0)


def _gather(tokens_t, table_lin, seq):
    mesh = plsc.VectorSubcoreMesh(core_axis_name="c", subcore_axis_name="s")
    return pl.kernel(
        _gather_body,
        out_type=jax.ShapeDtypeStruct((seq, EMB // 8, NW, 8 * BW), jnp.float32),
        mesh=mesh,
        compiler_params=pltpu.CompilerParams(
            use_tc_tiling_on_sc=False, needs_layout_passes=False
        ),
        scratch_types=[
            pltpu.VMEM((seq, BW), jnp.int32),
            pltpu.VMEM((BW, EMB), jnp.float32),
            pltpu.VMEM((BW, EMB), jnp.float32),
            pltpu.VMEM((EMB // 8, 8 * BW), jnp.float32),
            pltpu.SemaphoreType.DMA,
            pltpu.SemaphoreType.DMA,
        ],
    )(tokens_t, table_lin)


def kernel(tokens, table):
    b, s = tokens.shape
    vocab, emb = table.shape
    tokens_t = tokens.T.astype(jnp.int32)
    table_t = table.T
    table_lin = _transpose_scale(table_t).reshape(vocab, emb)
    p = _gather(tokens_t, table_lin, s)
    p5 = p.reshape(s, emb // 8, NW, 8, BW)
    return p5.transpose(2, 4, 0, 1, 3).reshape(b, s, emb)


# speedup vs baseline: 1.3087x; 1.3087x over previous
import math

import jax
import jax.numpy as jnp
from jax import lax
from jax.experimental import pallas as pl
from jax.experimental.pallas import tpu as pltpu
from jax.experimental.pallas import tpu_sc as plsc

EMB = 64
SCALE = math.sqrt(EMB)

NC = 2
NS = 16
NW = NC * NS

BW = 128
TB = 512


def _transpose_body(tab_t_hbm, out_hbm, sb0, sb1, tb_v, tail_s, tail_t, sem0, sem1):
    vocab = tab_t_hbm.shape[1]
    n_full = vocab // TB
    per_w = (n_full + NW - 1) // NW
    wid = lax.axis_index("s") * NC + lax.axis_index("c")

    lane64 = lax.shift_left(lax.iota(jnp.int32, 16), 6)

    def v0_of(i):
        m = jnp.minimum(wid + NW * i, n_full - 1)
        return pl.multiple_of(m * TB, TB)

    def fire(i, sb, sem):
        pltpu.async_copy(tab_t_hbm.at[:, pl.ds(v0_of(i), TB)], sb, sem)

    def drain(i, sb, sem):
        pltpu.make_async_copy(
            tab_t_hbm.at[:, pl.ds(v0_of(i), TB)], sb, sem
        ).wait()

    def transpose_block(i, sb):
        @plsc.parallel_loop(0, EMB * (TB // 16), 1, unroll=8)
        def _(k):
            e = jnp.bitwise_and(k, EMB - 1)
            g = lax.shift_right_logical(k, 6)
            x = sb[e, pl.ds(g * 16, 16)] * SCALE
            plsc.store_scatter(tb_v, [lane64 + lax.shift_left(g, 10) + e], x)

        pltpu.sync_copy(tb_v, out_hbm.at[pl.ds(v0_of(i) * EMB, TB * EMB)])

    fire(0, sb0, sem0)

    def pair(p, _):
        i0 = 2 * p
        fire(i0 + 1, sb1, sem1)
        drain(i0, sb0, sem0)
        transpose_block(i0, sb0)
        nxt = jnp.where(i0 + 2 < per_w, i0 + 2, 0)
        fire(nxt, sb0, sem0)
        drain(i0 + 1, sb1, sem1)
        transpose_block(i0 + 1, sb1)
        return _

    lax.fori_loop(0, per_w // 2, pair, None)
    drain(0, sb0, sem0)

    v0t = n_full * TB
    pltpu.sync_copy(tab_t_hbm.at[:, pl.ds(v0t, EMB)], tail_s)

    @plsc.parallel_loop(0, EMB * (EMB // 16), 1, unroll=8)
    def _(k):
        e = jnp.bitwise_and(k, EMB - 1)
        g = lax.shift_right_logical(k, 6)
        x = tail_s[e, pl.ds(g * 16, 16)] * SCALE
        plsc.store_scatter(tail_t, [lane64 + lax.shift_left(g, 10) + e], x)
    pltpu.sync_copy(tail_t, out_hbm.at[pl.ds(v0t * EMB, EMB * EMB)])


def _transpose_scale(table_t):
    emb, vocab = table_t.shape
    mesh = plsc.VectorSubcoreMesh(core_axis_name="c", subcore_axis_name="s")
    return pl.kernel(
        _transpose_body,
        out_type=jax.ShapeDtypeStruct((vocab * emb,), jnp.float32),
        mesh=mesh,
        compiler_params=pltpu.CompilerParams(needs_layout_passes=False),
        scratch_types=[
            pltpu.VMEM((EMB, TB), jnp.float32),
            pltpu.VMEM((EMB, TB), jnp.float32),
            pltpu.VMEM((TB * EMB,), jnp.float32),
            pltpu.VMEM((EMB, EMB), jnp.float32),
            pltpu.VMEM((EMB * EMB,), jnp.float32),
            pltpu.SemaphoreType.DMA,
            pltpu.SemaphoreType.DMA,
        ],
    )(table_t)


def _gather_body(tokens_t_hbm, table_hbm, out_hbm, idx_all, rb0, rb1, tr, gs0, gs1):
    seq = tokens_t_hbm.shape[0]
    wid = lax.axis_index("s") * NC + lax.axis_index("c")

    pltpu.sync_copy(tokens_t_hbm.at[:, pl.ds(wid * BW, BW)], idx_all)

    lane = lax.iota(jnp.int32, 16)
    rsh3 = lax.shift_right_logical(lane, 3)
    cvec = lax.shift_left(jnp.bitwise_and(lane, 7), 7)

    def fire(t, rb, sem):
        pltpu.async_copy(table_hbm.at[idx_all.at[t]], rb, sem)

    def drain(t, rb, sem):
        pltpu.make_async_copy(table_hbm.at[idx_all.at[t]], rb, sem).wait()

    def process(t, rb):
        @plsc.parallel_loop(0, BW * (EMB // 16), 1, unroll=8)
        def _(k):
            bm = lax.shift_right_logical(k, 2)
            j = jnp.bitwise_and(k, 3)
            x = rb[bm, pl.ds(j * 16, 16)]
            plsc.store_scatter(tr, [lax.shift_left(j, 1) + rsh3, cvec + bm], x)

        for i in range(8):
            pltpu.sync_copy(tr.at[i], out_hbm.at[t, i, wid])

    fire(0, rb0, gs0)

    def pair(p, _):
        t0 = 2 * p
        fire(t0 + 1, rb1, gs1)
        drain(t0, rb0, gs0)
        process(t0, rb0)
        nxt = jnp.where(t0 + 2 < seq, t0 + 2, 0)
        fire(nxt, rb0, gs0)
        drain(t0 + 1, rb1, gs1)
        process(t0 + 1, rb1)
        return _

    lax.fori_loop(0, seq // 2, pair, None)
    drain(0, rb0, gs0)


def _gather(tokens_t, table_lin, seq):
    mesh = plsc.VectorSubcoreMesh(core_axis_name="c", subcore_axis_name="s")
    return pl.kernel(
        _gather_body,
        out_type=jax.ShapeDtypeStruct((seq, EMB // 8, NW, 8 * BW), jnp.float32),
        mesh=mesh,
        compiler_params=pltpu.CompilerParams(
            use_tc_tiling_on_sc=False, needs_layout_passes=False
        ),
        scratch_types=[
            pltpu.VMEM((seq, BW), jnp.int32),
            pltpu.VMEM((BW, EMB), jnp.float32),
            pltpu.VMEM((BW, EMB), jnp.float32),
            pltpu.VMEM((EMB // 8, 8 * BW), jnp.float32),
            pltpu.SemaphoreType.DMA,
            pltpu.SemaphoreType.DMA,
        ],
    )(tokens_t, table_lin)


def kernel(tokens, table):
    b, s = tokens.shape
    vocab, emb = table.shape
    tokens_t = tokens.T.astype(jnp.int32)
    table_t = table.T
    table_lin = _transpose_scale(table_t).reshape(vocab, emb)
    p = _gather(tokens_t, table_lin, s)
    p5 = p.reshape(s, emb // 8, NW, 8, BW)
    return p5.transpose(2, 4, 0, 1, 3).reshape(b, s, emb)
